# Initial kernel scaffold; baseline (speedup 1.0000x reference)
import jax
import jax.numpy as jnp
from jax.experimental import pallas as pl


def kernel(x, Wg, bg, Wn, bn, noise, top_k):
    raise NotImplementedError("write your pallas kernel here")


# speedup vs baseline: 1.6209x; 1.6209x over previous
import functools

import jax
import jax.numpy as jnp
from jax.experimental import pallas as pl
from jax.experimental.pallas import tpu as pltpu

_BB = 1024
_K = 8


def _body(x_ref, w_ref, b_ref, n_ref, wout_ref, lout_ref, vout_ref, iout_ref,
          *, E):
    acc = jnp.dot(x_ref[...], w_ref[...], preferred_element_type=jnp.float32)
    acc = acc + b_ref[...]
    clean = acc[:, :E]
    raw = acc[:, E:]
    std = jnp.maximum(raw, 0.0) + jnp.log1p(jnp.exp(-jnp.abs(raw)))
    logits = clean + n_ref[...] * std
    lout_ref[...] = logits
    m = jnp.max(logits, axis=-1, keepdims=True)
    e = jnp.exp(logits - m)
    s = jnp.sum(e, axis=-1, keepdims=True)
    weights = e / s
    wout_ref[...] = weights
    cols = jax.lax.broadcasted_iota(jnp.int32, weights.shape, 1)
    work = weights
    vals, idxs = [], []
    for _ in range(_K):
        mx = jnp.max(work, axis=-1, keepdims=True)
        am = jnp.argmax(work, axis=-1).astype(jnp.int32)[:, None]
        vals.append(mx)
        idxs.append(am)
        work = jnp.where(cols == am, -1.0, work)
    vout_ref[...] = jnp.concatenate(vals, axis=1)
    iout_ref[...] = jnp.concatenate(idxs, axis=1)


def kernel(x, Wg, bg, Wn, bn, noise, top_k):
    B, D = x.shape
    E = Wg.shape[0]
    W = jnp.concatenate([Wg, Wn], axis=0).T
    b2 = jnp.concatenate([bg, bn])[None, :]
    grid = (B // _BB,)
    out = pl.pallas_call(
        functools.partial(_body, E=E),
        grid=grid,
        in_specs=[
            pl.BlockSpec((_BB, D), lambda i: (i, 0)),
            pl.BlockSpec((D, 2 * E), lambda i: (0, 0)),
            pl.BlockSpec((1, 2 * E), lambda i: (0, 0)),
            pl.BlockSpec((_BB, E), lambda i: (i, 0)),
        ],
        out_specs=[
            pl.BlockSpec((_BB, E), lambda i: (i, 0)),
            pl.BlockSpec((_BB, E), lambda i: (i, 0)),
            pl.BlockSpec((_BB, _K), lambda i: (i, 0)),
            pl.BlockSpec((_BB, _K), lambda i: (i, 0)),
        ],
        out_shape=[
            jax.ShapeDtypeStruct((B, E), jnp.float32),
            jax.ShapeDtypeStruct((B, E), jnp.float32),
            jax.ShapeDtypeStruct((B, _K), jnp.float32),
            jax.ShapeDtypeStruct((B, _K), jnp.int32),
        ],
        compiler_params=pltpu.CompilerParams(
            dimension_semantics=("parallel",)),
    )(x, W, b2, noise)
    weights, logits, top_k_vals, top_k_indices = out
    return (weights, logits, top_k_vals, top_k_indices)


# speedup vs baseline: 1.6441x; 1.0144x over previous
import functools

import jax
import jax.numpy as jnp
from jax.experimental import pallas as pl
from jax.experimental.pallas import tpu as pltpu

_BB = 1024
_K = 8
_NEG = -3.0e38


def _body(x_ref, w_ref, b_ref, n_ref, wout_ref, lout_ref, vout_ref, iout_ref,
          *, E):
    acc = jnp.dot(x_ref[...], w_ref[...], preferred_element_type=jnp.float32)
    acc = acc + b_ref[...]
    clean = acc[:, :E]
    raw = acc[:, E:]
    std = jnp.maximum(raw, 0.0) + jnp.log1p(jnp.exp(-jnp.abs(raw)))
    logits = clean + n_ref[...] * std
    lout_ref[...] = logits
    cols = jax.lax.broadcasted_iota(jnp.int32, logits.shape, 1)
    work = logits
    mxs, idxs = [], []
    for _ in range(_K):
        mx = jnp.max(work, axis=-1, keepdims=True)
        am = jnp.argmax(work, axis=-1).astype(jnp.int32)[:, None]
        mxs.append(mx)
        idxs.append(am)
        work = jnp.where(cols == am, _NEG, work)
    m = mxs[0]
    e = jnp.exp(logits - m)
    s = jnp.sum(e, axis=-1, keepdims=True)
    inv_s = 1.0 / s
    wout_ref[...] = e * inv_s
    tl = jnp.concatenate(mxs, axis=1)
    vout_ref[...] = jnp.exp(tl - m) * inv_s
    iout_ref[...] = jnp.concatenate(idxs, axis=1)


def kernel(x, Wg, bg, Wn, bn, noise, top_k):
    B, D = x.shape
    E = Wg.shape[0]
    W = jnp.concatenate([Wg, Wn], axis=0).T
    b2 = jnp.concatenate([bg, bn])[None, :]
    grid = (B // _BB,)
    out = pl.pallas_call(
        functools.partial(_body, E=E),
        grid=grid,
        in_specs=[
            pl.BlockSpec((_BB, D), lambda i: (i, 0)),
            pl.BlockSpec((D, 2 * E), lambda i: (0, 0)),
            pl.BlockSpec((1, 2 * E), lambda i: (0, 0)),
            pl.BlockSpec((_BB, E), lambda i: (i, 0)),
        ],
        out_specs=[
            pl.BlockSpec((_BB, E), lambda i: (i, 0)),
            pl.BlockSpec((_BB, E), lambda i: (i, 0)),
            pl.BlockSpec((_BB, _K), lambda i: (i, 0)),
            pl.BlockSpec((_BB, _K), lambda i: (i, 0)),
        ],
        out_shape=[
            jax.ShapeDtypeStruct((B, E), jnp.float32),
            jax.ShapeDtypeStruct((B, E), jnp.float32),
            jax.ShapeDtypeStruct((B, _K), jnp.float32),
            jax.ShapeDtypeStruct((B, _K), jnp.int32),
        ],
        compiler_params=pltpu.CompilerParams(
            dimension_semantics=("parallel",)),
    )(x, W, b2, noise)
    weights, logits, top_k_vals, top_k_indices = out
    return (weights, logits, top_k_vals, top_k_indices)
